# Initial kernel scaffold; baseline (speedup 1.0000x reference)
import jax
import jax.numpy as jnp
from jax.experimental import pallas as pl


def kernel(x, emb, W1, b1, W2, b2):
    raise NotImplementedError("write your pallas kernel here")


# speedup vs baseline: 2.8473x; 2.8473x over previous
import functools

import jax
import jax.numpy as jnp
from jax import lax
from jax.experimental import pallas as pl
from jax.experimental.pallas import tpu as pltpu
from jax.experimental.pallas import tpu_sc as plsc


def _make_sc_pool(B, S, D, V):
    info = plsc.get_sparse_core_info()
    NC, NS = info.num_cores, info.num_subcores
    NW = NC * NS
    rows_per_w = B // NW
    GRP = 8
    n_groups = rows_per_w // GRP
    S_HI = 128
    S_LO = S - S_HI

    mesh = plsc.VectorSubcoreMesh(core_axis_name="c", subcore_axis_name="s")

    @functools.partial(
        pl.kernel,
        out_type=jax.ShapeDtypeStruct((B, D), jnp.float32),
        mesh=mesh,
        scratch_types=[
            pltpu.VMEM((2 * GRP, 128), jnp.int32),
            pltpu.VMEM((2, S, D), jnp.float32),
            pltpu.VMEM((GRP, D), jnp.float32),
            pltpu.SemaphoreType.DMA,
            pltpu.SemaphoreType.DMA,
        ],
        compiler_params=pltpu.CompilerParams(use_tc_tiling_on_sc=False),
    )
    def sc_pool(x2d, emb, out, idx_v, gbuf, acc_v, sem0, sem1):
        wid = lax.axis_index("s") * NC + lax.axis_index("c")
        row0 = wid * rows_per_w
        sems = (sem0, sem1)

        def fire(r8, slot):
            sem = sems[slot]
            c0 = pltpu.async_copy(
                emb.at[idx_v.at[2 * r8]],
                gbuf.at[slot, pl.ds(0, S_HI)], sem)
            c1 = pltpu.async_copy(
                emb.at[idx_v.at[2 * r8 + 1, pl.ds(0, S_LO)]],
                gbuf.at[slot, pl.ds(S_HI, S_LO)], sem)
            return (c0, c1)

        def accum(slot, r8):
            zero = jnp.zeros((16,), jnp.float32)

            def body(s, acc):
                return tuple(acc[d] + gbuf[slot, s, pl.ds(d * 16, 16)]
                             for d in range(D // 16))

            acc = lax.fori_loop(0, S, body, (zero,) * (D // 16))
            for d in range(D // 16):
                acc_v[r8, pl.ds(d * 16, 16)] = acc[d]

        def group(k, _):
            base = row0 + k * GRP
            pltpu.sync_copy(x2d.at[pl.ds(base * 2, 2 * GRP)], idx_v)
            pend = fire(0, 0)
            for r8 in range(GRP):
                slot = r8 % 2
                nxt = fire(r8 + 1, 1 - slot) if r8 + 1 < GRP else None
                pend[0].wait()
                pend[1].wait()
                accum(slot, r8)
                pend = nxt
            pltpu.sync_copy(acc_v, out.at[pl.ds(base, GRP)])
            return _

        lax.fori_loop(0, n_groups, group, None)

    return sc_pool


def _mlp_body(x_ref, ps_ref, w1_ref, b1_ref, w2_ref, b2_ref, o_ref):
    xb = x_ref[...]
    cnt = jnp.sum((xb != 0).astype(jnp.float32), axis=1, keepdims=True)
    cnt = jnp.maximum(cnt, 1.0)
    pooled = ps_ref[...] / cnt
    h = jnp.dot(pooled, w1_ref[...], preferred_element_type=jnp.float32)
    h = jnp.maximum(h + b1_ref[...], 0.0)
    o_ref[...] = (jnp.dot(h, w2_ref[...], preferred_element_type=jnp.float32)
                  + b2_ref[...])


def _mlp_call(x, pooled_sum, W1, b1, W2, b2):
    B, S = x.shape
    D = W1.shape[0]
    H = W1.shape[1]
    C = W2.shape[1]
    BLK = 2048
    grid = (B // BLK,)
    return pl.pallas_call(
        _mlp_body,
        grid=grid,
        in_specs=[
            pl.BlockSpec((BLK, S), lambda i: (i, 0)),
            pl.BlockSpec((BLK, D), lambda i: (i, 0)),
            pl.BlockSpec((D, H), lambda i: (0, 0)),
            pl.BlockSpec((1, H), lambda i: (0, 0)),
            pl.BlockSpec((H, C), lambda i: (0, 0)),
            pl.BlockSpec((1, C), lambda i: (0, 0)),
        ],
        out_specs=pl.BlockSpec((BLK, C), lambda i: (i, 0)),
        out_shape=jax.ShapeDtypeStruct((B, C), jnp.float32),
    )(x, pooled_sum, W1, b1.reshape(1, H), W2, b2.reshape(1, C))


def kernel(x, emb, W1, b1, W2, b2):
    B, S = x.shape
    V, D = emb.shape
    x = x.astype(jnp.int32)
    x_pad = jnp.pad(x, ((0, 0), (0, 256 - S)))
    x2d = x_pad.reshape(B * 2, 128)
    pooled_sum = _make_sc_pool(B, S, D, V)(x2d, emb)
    return _mlp_call(x, pooled_sum, W1, b1, W2, b2)
